# baseline (speedup 1.0000x reference)
import functools

import jax
import jax.numpy as jnp
from jax import lax
from jax.experimental import pallas as pl
from jax.experimental.pallas import tpu as pltpu
from jax.experimental.pallas import tpu_sc as plsc


def kernel(uEmbeds, iEmbeds, ancs, poss, negs):
    B = ancs.shape[0]
    D = uEmbeds.shape[1]
    info = plsc.get_sparse_core_info()
    nc, ns = info.num_cores, info.num_subcores
    nw = nc * ns
    b_per_w = B // nw
    mesh = plsc.VectorSubcoreMesh(core_axis_name="c", subcore_axis_name="s")

    @functools.partial(
        pl.kernel,
        mesh=mesh,
        out_type=jax.ShapeDtypeStruct((B,), jnp.float32),
        compiler_params=pltpu.CompilerParams(
            needs_layout_passes=False, use_tc_tiling_on_sc=False),
        scratch_types=[
            pltpu.VMEM((b_per_w,), jnp.int32),
            pltpu.VMEM((b_per_w,), jnp.int32),
            pltpu.VMEM((b_per_w,), jnp.int32),
            pltpu.VMEM((b_per_w, D), jnp.float32),
            pltpu.VMEM((b_per_w, D), jnp.float32),
            pltpu.VMEM((b_per_w, D), jnp.float32),
            pltpu.VMEM((b_per_w,), jnp.float32),
            pltpu.SemaphoreType.DMA,
            pltpu.SemaphoreType.DMA,
            pltpu.SemaphoreType.DMA,
        ],
    )
    def mf_scores(u_hbm, i_hbm, anc_hbm, pos_hbm, neg_hbm, out_hbm,
                  anc_idx, pos_idx, neg_idx, anc_v, pos_v, neg_v, out_v,
                  sem_a, sem_p, sem_n):
        wid = lax.axis_index("s") * nc + lax.axis_index("c")
        base = wid * b_per_w
        pltpu.sync_copy(anc_hbm.at[pl.ds(base, b_per_w)], anc_idx)
        pltpu.sync_copy(pos_hbm.at[pl.ds(base, b_per_w)], pos_idx)
        pltpu.sync_copy(neg_hbm.at[pl.ds(base, b_per_w)], neg_idx)
        ca = pltpu.async_copy(u_hbm.at[anc_idx], anc_v, sem_a)
        cp = pltpu.async_copy(i_hbm.at[pos_idx], pos_v, sem_p)
        cn = pltpu.async_copy(i_hbm.at[neg_idx], neg_v, sem_n)
        ca.wait()
        cp.wait()
        cn.wait()

        lane = lax.iota(jnp.int32, 16)

        def chunk_body(j, carry):
            rows = j * 16 + lane
            acc = jnp.zeros((16,), jnp.float32)
            for col_i in range(D):
                col = jnp.full((16,), col_i, jnp.int32)
                a = plsc.load_gather(anc_v, [rows, col])
                p = plsc.load_gather(pos_v, [rows, col])
                n = plsc.load_gather(neg_v, [rows, col])
                acc = acc + a * (p - n)
            out_v[pl.ds(j * 16, 16)] = acc
            return carry

        lax.fori_loop(0, b_per_w // 16, chunk_body, 0)
        pltpu.sync_copy(out_v, out_hbm.at[pl.ds(base, b_per_w)])

    return mf_scores(uEmbeds, iEmbeds, ancs, poss, negs)
